# baseline (speedup 1.0000x reference)
import jax
import jax.numpy as jnp
from jax.experimental import pallas as pl
from jax.experimental.pallas import tpu as pltpu

_B = 8
_H = 512
_W = 512
_R = 128
_N = _B * _H * _W


def _loss_kernel(semi_ref, gt_ref, out_ref):
    b = pl.program_id(0)
    k = pl.program_id(1)
    nb = pl.num_programs(0)
    nk = pl.num_programs(1)

    s0 = semi_ref[0]
    s1 = semi_ref[1]
    g = gt_ref[0]
    part = jnp.sum(g * jnp.log(s0) + (1.0 - g) * jnp.log(s1))

    @pl.when((b == 0) & (k == 0))
    def _init():
        out_ref[0, 0] = 0.0

    out_ref[0, 0] += part

    @pl.when((b == nb - 1) & (k == nk - 1))
    def _finalize():
        out_ref[0, 0] = out_ref[0, 0] * (-1.0 / _N)


def kernel(semi, gt_score, desc):
    del desc
    semi2 = semi.reshape(_B * 2, _H, _W)
    nk = _H // _R
    out = pl.pallas_call(
        _loss_kernel,
        grid=(_B, nk),
        in_specs=[
            pl.BlockSpec((2, _R, _W), lambda b, k: (b, k, 0)),
            pl.BlockSpec((1, _R, _W), lambda b, k: (b, k, 0)),
        ],
        out_specs=pl.BlockSpec(
            (1, 1), lambda b, k: (0, 0), memory_space=pltpu.SMEM
        ),
        out_shape=jax.ShapeDtypeStruct((1, 1), jnp.float32),
    )(semi2, gt_score)
    return out[0, 0]


# speedup vs baseline: 1.0483x; 1.0483x over previous
import jax
import jax.numpy as jnp
from jax.experimental import pallas as pl
from jax.experimental.pallas import tpu as pltpu

_B = 8
_H = 512
_W = 512
_R = 256
_N = _B * _H * _W


def _loss_kernel(semi_ref, gt_ref, out_ref, acc_ref):
    b = pl.program_id(0)
    k = pl.program_id(1)
    c = pl.program_id(2)
    nb = pl.num_programs(0)
    nk = pl.num_programs(1)

    @pl.when((b == 0) & (k == 0) & (c == 0))
    def _init():
        acc_ref[...] = jnp.zeros_like(acc_ref)

    g = gt_ref[0, 0]
    cf = c.astype(jnp.float32)
    w = cf + (1.0 - 2.0 * cf) * g
    acc_ref[...] += w * jnp.log(semi_ref[0, 0])

    @pl.when((b == nb - 1) & (k == nk - 1) & (c == 1))
    def _finalize():
        out_ref[0, 0] = jnp.sum(acc_ref[...]) * (-1.0 / _N)


def kernel(semi, gt_score, desc):
    del desc
    nk = _H // _R
    out = pl.pallas_call(
        _loss_kernel,
        grid=(_B, nk, 2),
        in_specs=[
            pl.BlockSpec((1, 1, _R, _W), lambda b, k, c: (b, c, k, 0)),
            pl.BlockSpec((1, _R, _W), lambda b, k, c: (b, k, 0)),
        ],
        out_specs=pl.BlockSpec(
            (1, 1), lambda b, k, c: (0, 0), memory_space=pltpu.SMEM
        ),
        out_shape=jax.ShapeDtypeStruct((1, 1), jnp.float32),
        scratch_shapes=[pltpu.VMEM((_R, _W), jnp.float32)],
    )(semi, gt_score)
    return out[0, 0]


# speedup vs baseline: 1.4278x; 1.3621x over previous
import jax
import jax.numpy as jnp
from jax import lax
from jax.experimental import pallas as pl
from jax.experimental.pallas import tpu as pltpu

_B = 8
_H = 512
_W = 512
_R = 256
_N = _B * _H * _W

_LN2 = 0.6931471805599453
_C = (
    -2.103426409714172,
    4.2211940777136086,
    -3.6748647207953504,
    2.252358585277505,
    -0.8650216851477687,
    0.1871757022552541,
    -0.01741407752406494,
)


def _poly_log(x):
    bits = lax.bitcast_convert_type(x, jnp.int32)
    e = ((bits >> 23) - 127).astype(jnp.float32)
    m = lax.bitcast_convert_type(
        (bits & jnp.int32(0x007FFFFF)) | jnp.int32(0x3F800000), jnp.float32
    )
    p = jnp.float32(_C[6])
    p = p * m + jnp.float32(_C[5])
    p = p * m + jnp.float32(_C[4])
    p = p * m + jnp.float32(_C[3])
    p = p * m + jnp.float32(_C[2])
    p = p * m + jnp.float32(_C[1])
    p = p * m + jnp.float32(_C[0])
    return e * jnp.float32(_LN2) + p


def _loss_kernel(semi_ref, gt_ref, out_ref, acc_ref):
    b = pl.program_id(0)
    k = pl.program_id(1)
    nb = pl.num_programs(0)
    nk = pl.num_programs(1)

    @pl.when((b == 0) & (k == 0))
    def _init():
        acc_ref[...] = jnp.zeros_like(acc_ref)

    g = gt_ref[0]
    l0 = _poly_log(semi_ref[0])
    l1 = jnp.log(semi_ref[1])
    acc_ref[...] += l1 + g * (l0 - l1)

    @pl.when((b == nb - 1) & (k == nk - 1))
    def _finalize():
        out_ref[0, 0] = jnp.sum(acc_ref[...]) * (-1.0 / _N)


def kernel(semi, gt_score, desc):
    del desc
    semi2 = semi.reshape(_B * 2, _H, _W)
    nk = _H // _R
    out = pl.pallas_call(
        _loss_kernel,
        grid=(_B, nk),
        in_specs=[
            pl.BlockSpec((2, _R, _W), lambda b, k: (b, k, 0)),
            pl.BlockSpec((1, _R, _W), lambda b, k: (b, k, 0)),
        ],
        out_specs=pl.BlockSpec(
            (1, 1), lambda b, k: (0, 0), memory_space=pltpu.SMEM
        ),
        out_shape=jax.ShapeDtypeStruct((1, 1), jnp.float32),
        scratch_shapes=[pltpu.VMEM((_R, _W), jnp.float32)],
    )(semi2, gt_score)
    return out[0, 0]


# speedup vs baseline: 1.5126x; 1.0594x over previous
import jax
import jax.numpy as jnp
from jax import lax
from jax.experimental import pallas as pl
from jax.experimental.pallas import tpu as pltpu

_B = 8
_H = 512
_W = 512
_R = 256
_RP = 224
_N = _B * _H * _W

_LN2 = 0.6931471805599453
_K1 = _LN2 / (1 << 23)
_C = (
    -1.2436125623821535 - 127.0 * _LN2,
    2.8209401164401315,
    -2.4400297626142167,
    1.116090026832197,
    -0.28382684778207107,
    0.030449004538664757,
)


def _poly_log(x):
    bits = lax.bitcast_convert_type(x, jnp.int32)
    bf = bits.astype(jnp.float32)
    m = lax.bitcast_convert_type(
        (bits & jnp.int32(0x007FFFFF)) | jnp.int32(0x3F800000), jnp.float32
    )
    p = jnp.float32(_C[5])
    p = p * m + jnp.float32(_C[4])
    p = p * m + jnp.float32(_C[3])
    p = p * m + jnp.float32(_C[2])
    p = p * m + jnp.float32(_C[1])
    p = p * m + jnp.float32(_C[0])
    return bf * jnp.float32(_K1) + p


def _loss_kernel(semi_ref, gt_ref, out_ref, acc_ref):
    b = pl.program_id(0)
    k = pl.program_id(1)
    nb = pl.num_programs(0)
    nk = pl.num_programs(1)

    @pl.when((b == 0) & (k == 0))
    def _init():
        acc_ref[...] = jnp.zeros_like(acc_ref)

    l1 = jnp.log(semi_ref[1])
    l0a = _poly_log(semi_ref[0, :_RP])
    l0b = jnp.log(semi_ref[0, _RP:])
    ga = gt_ref[0, :_RP]
    gb = gt_ref[0, _RP:]
    acc_ref[:_RP] += l1[:_RP] + ga * (l0a - l1[:_RP])
    acc_ref[_RP:] += l1[_RP:] + gb * (l0b - l1[_RP:])

    @pl.when((b == nb - 1) & (k == nk - 1))
    def _finalize():
        out_ref[0, 0] = jnp.sum(acc_ref[...]) * (-1.0 / _N)


def kernel(semi, gt_score, desc):
    del desc
    semi2 = semi.reshape(_B * 2, _H, _W)
    nk = _H // _R
    out = pl.pallas_call(
        _loss_kernel,
        grid=(_B, nk),
        in_specs=[
            pl.BlockSpec((2, _R, _W), lambda b, k: (b, k, 0)),
            pl.BlockSpec((1, _R, _W), lambda b, k: (b, k, 0)),
        ],
        out_specs=pl.BlockSpec(
            (1, 1), lambda b, k: (0, 0), memory_space=pltpu.SMEM
        ),
        out_shape=jax.ShapeDtypeStruct((1, 1), jnp.float32),
        scratch_shapes=[pltpu.VMEM((_R, _W), jnp.float32)],
    )(semi2, gt_score)
    return out[0, 0]


# speedup vs baseline: 1.5474x; 1.0230x over previous
import jax
import jax.numpy as jnp
from jax import lax
from jax.experimental import pallas as pl
from jax.experimental.pallas import tpu as pltpu

_B = 8
_H = 512
_W = 512
_R = 256
_RP = 224
_N = _B * _H * _W

_LN2 = 0.6931471805599453
_K1 = _LN2 / (1 << 23)
_C = (
    -0.7936123702332034 - 127.0 * _LN2,
    1.4067271006724422,
    -0.7203588649841475,
    0.10774685617806046,
)


def _poly_log(x):
    bits = lax.bitcast_convert_type(x, jnp.int32)
    bf = bits.astype(jnp.float32)
    m = lax.bitcast_convert_type(
        (bits & jnp.int32(0x007FFFFF)) | jnp.int32(0x3F800000), jnp.float32
    )
    p = jnp.float32(_C[3])
    p = p * m + jnp.float32(_C[2])
    p = p * m + jnp.float32(_C[1])
    p = p * m + jnp.float32(_C[0])
    return bf * jnp.float32(_K1) + p


def _loss_kernel(semi_ref, gt_ref, out_ref, acc_ref):
    b = pl.program_id(0)
    k = pl.program_id(1)
    nb = pl.num_programs(0)
    nk = pl.num_programs(1)

    @pl.when((b == 0) & (k == 0))
    def _init():
        acc_ref[...] = jnp.zeros_like(acc_ref)

    l1 = jnp.log(semi_ref[1])
    l0a = _poly_log(semi_ref[0, :_RP])
    l0b = jnp.log(semi_ref[0, _RP:])
    ga = gt_ref[0, :_RP]
    gb = gt_ref[0, _RP:]
    acc_ref[:_RP] += l1[:_RP] + ga * (l0a - l1[:_RP])
    acc_ref[_RP:] += l1[_RP:] + gb * (l0b - l1[_RP:])

    @pl.when((b == nb - 1) & (k == nk - 1))
    def _finalize():
        out_ref[0, 0] = jnp.sum(acc_ref[...]) * (-1.0 / _N)


def kernel(semi, gt_score, desc):
    del desc
    semi2 = semi.reshape(_B * 2, _H, _W)
    nk = _H // _R
    out = pl.pallas_call(
        _loss_kernel,
        grid=(_B, nk),
        in_specs=[
            pl.BlockSpec((2, _R, _W), lambda b, k: (b, k, 0)),
            pl.BlockSpec((1, _R, _W), lambda b, k: (b, k, 0)),
        ],
        out_specs=pl.BlockSpec(
            (1, 1), lambda b, k: (0, 0), memory_space=pltpu.SMEM
        ),
        out_shape=jax.ShapeDtypeStruct((1, 1), jnp.float32),
        scratch_shapes=[pltpu.VMEM((_R, _W), jnp.float32)],
    )(semi2, gt_score)
    return out[0, 0]
